# baseline (speedup 1.0000x reference)
import jax
import jax.numpy as jnp
from jax import lax
from jax.experimental import pallas as pl
from jax.experimental.pallas import tpu as pltpu
from jax.experimental.pallas import tpu_sc as plsc

FEAT_H = 16
FEAT_W = 16
HW = FEAT_H * FEAT_W
GAUSS_TH = 0.6
EXPAND = 1.0
SIGMA = 1.6
BIN_OFF = 0.5
RADIUS = 1.0
NEG_LOG_TH = 0.5108255840295616
TWO_SIG2 = 2.0 * SIGMA ** 2
IMGS = 2
N_PER_IMG = 512
G_PER_IMG = 64

_NTILES = 32
_RPT = 64


def _take(v, idx):
    return lax.gather(
        v, idx[:, None],
        lax.GatherDimensionNumbers(offset_dims=(), collapsed_slice_dims=(0,),
                                   start_index_map=(0,)),
        (1,), mode=lax.GatherScatterMode.PROMISE_IN_BOUNDS)


def _weights_sc(x1_h, y1_h, x2_h, y2_h, flag_h, gid_h, kx_h, ky_h, kv_h,
                clsw_h, regw_h,
                x1_v, y1_v, x2_v, y2_v, flag_v, gid_v, kx_v, ky_v, kv_v,
                clsw_v, regw_v):
    wid = lax.axis_index("s") * 2 + lax.axis_index("c")
    base = wid * _RPT
    img = wid // (N_PER_IMG // _RPT)
    pltpu.sync_copy(x1_h.at[pl.ds(base, _RPT)], x1_v)
    pltpu.sync_copy(y1_h.at[pl.ds(base, _RPT)], y1_v)
    pltpu.sync_copy(x2_h.at[pl.ds(base, _RPT)], x2_v)
    pltpu.sync_copy(y2_h.at[pl.ds(base, _RPT)], y2_v)
    pltpu.sync_copy(flag_h.at[pl.ds(base, _RPT)], flag_v)
    pltpu.sync_copy(gid_h.at[pl.ds(base, _RPT)], gid_v)
    pltpu.sync_copy(kx_h.at[pl.ds(img * G_PER_IMG, G_PER_IMG)], kx_v)
    pltpu.sync_copy(ky_h.at[pl.ds(img * G_PER_IMG, G_PER_IMG)], ky_v)
    pltpu.sync_copy(kv_h.at[pl.ds(img * G_PER_IMG, G_PER_IMG)], kv_v)

    iota16 = lax.broadcasted_iota(jnp.int32, (16,), 0)
    zero16 = jnp.zeros((16,), jnp.float32)
    binc05 = iota16.astype(jnp.float32) + BIN_OFF

    kxc = [kx_v[pl.ds(c * 16, 16)] for c in range(4)]
    kyc = [ky_v[pl.ds(c * 16, 16)] for c in range(4)]
    kvc = [kv_v[pl.ds(c * 16, 16)] for c in range(4)]

    def tab_gather(chunks, lo, hi):
        acc = zero16
        for c in range(4):
            acc = jnp.where(hi == c, _take(chunks[c], lo), acc)
        return acc

    for g in range(_RPT // 16):
        sl = pl.ds(g * 16, 16)
        x1 = x1_v[sl]
        y1 = y1_v[sl]
        x2 = x2_v[sl]
        y2 = y2_v[sl]
        cx = (x1 + x2) * 0.5
        cy = (y1 + y2) * 0.5
        w = (x2 - x1 + 1.0) * EXPAND
        h = (y2 - y1 + 1.0) * EXPAND
        bx1 = cx - (w - 1.0) * 0.5
        by1 = cy - (h - 1.0) * 0.5
        bx2 = cx + (w - 1.0) * 0.5
        by2 = cy + (h - 1.0) * 0.5

        gid16 = gid_v[sl]
        lo = gid16 & 15
        hi = lax.shift_right_logical(gid16, 4)
        kx = tab_gather(kxc, lo, hi)
        ky = tab_gather(kyc, lo, hi)
        kv = tab_gather(kvc, lo, hi)

        sx = FEAT_W / (bx2 - bx1 + 1.0)
        sy = FEAT_H / (by2 - by1 + 1.0)
        x0v = (kx - bx1) * sx
        y0v = (ky - by1) * sy
        pvv = jnp.where((flag_v[sl] > 0) & (kv != 0.0), zero16 + 1.0, zero16)

        for p in range(16):
            sp = iota16 * 0 + p
            bx0 = _take(x0v, sp)
            by0 = _take(y0v, sp)
            pvb = _take(pvv, sp)
            dxv = binc05 - bx0
            argx = dxv * dxv / TWO_SIG2
            dyv = binc05 - by0
            ayv = dyv * dyv / TWO_SIG2
            row = g * 16 + p
            macc = zero16
            for j in range(FEAT_H):
                argj = argx + _take(ayv, iota16 * 0 + j)
                rw = jnp.where(argj <= NEG_LOG_TH, pvb, zero16)
                macc = jnp.maximum(macc, rw)
                regw_v[row, pl.ds(j * 16, 16)] = rw
                regw_v[row, pl.ds(HW + j * 16, 16)] = rw
            for s in (1, 2, 4, 8):
                macc = jnp.maximum(macc, _take(macc, iota16 ^ s))
            for j in range(FEAT_H):
                clsw_v[row, pl.ds(j * 16, 16)] = macc

    n0 = (wid % (N_PER_IMG // _RPT)) * _RPT
    pltpu.sync_copy(clsw_v, clsw_h.at[img, pl.ds(n0, _RPT)])
    pltpu.sync_copy(regw_v, regw_h.at[img, pl.ds(n0, _RPT)])


def _label_kernel(boxes_ref, gt_ref, flag_ref, gid_ref, cls_ref, reg_ref):
    rows = IMGS * N_PER_IMG
    ng = IMGS * G_PER_IMG
    boxes = boxes_ref[...].reshape(rows, 4)
    gt = gt_ref[...].reshape(ng, 8)
    flag = flag_ref[...].reshape(rows, 1)
    gid = gid_ref[...].reshape(rows, 1)

    goff = (lax.broadcasted_iota(jnp.int32, (rows, 1), 0)
            // N_PER_IMG) * G_PER_IMG
    gslot = gid + goff
    onehot = (gslot == lax.broadcasted_iota(jnp.int32, (rows, ng), 1)
              ).astype(jnp.float32)
    matched = jnp.dot(onehot, gt, preferred_element_type=jnp.float32,
                      precision=lax.Precision.HIGHEST)

    x1 = boxes[:, 0:1]
    y1 = boxes[:, 1:2]
    x2 = boxes[:, 2:3]
    y2 = boxes[:, 3:4]
    cx = (x1 + x2) * 0.5
    cy = (y1 + y2) * 0.5
    w = (x2 - x1 + 1.0) * EXPAND
    h = (y2 - y1 + 1.0) * EXPAND
    bx1 = cx - (w - 1.0) * 0.5
    by1 = cy - (h - 1.0) * 0.5
    bx2 = cx + (w - 1.0) * 0.5
    by2 = cy + (h - 1.0) * 0.5

    kx = matched[:, 4:5]
    ky = matched[:, 5:6]
    kv = matched[:, 6:7]

    sx = FEAT_W / (bx2 - bx1 + 1.0)
    sy = FEAT_H / (by2 - by1 + 1.0)
    x0 = (kx - bx1) * sx
    y0 = (ky - by1) * sy

    col = lax.broadcasted_iota(jnp.int32, (rows, HW), 1)
    bin_x = (col % FEAT_W).astype(jnp.float32)
    bin_y = (col // FEAT_W).astype(jnp.float32)

    dx = bin_x + BIN_OFF - x0
    dy = bin_y + BIN_OFF - y0
    inv2s2 = 1.0 / TWO_SIG2
    arg = dx * dx * inv2s2 + dy * dy * inv2s2
    score = jnp.exp(-arg)
    keep = arg <= NEG_LOG_TH

    vis = kv != 0.0
    pos = flag > 0
    active = pos & vis & jnp.any(keep, axis=-1, keepdims=True)

    cls_ref[...] = jnp.where(active, score, -1.0).reshape(IMGS, N_PER_IMG, HW)

    m = pos & vis & keep
    off_x = (x0 - bin_x) / RADIUS
    off_y = (y0 - bin_y) / RADIUS
    zeros = jnp.zeros_like(score)
    reg = jnp.concatenate([jnp.where(m, off_x, zeros),
                           jnp.where(m, off_y, zeros)], axis=1)
    reg_ref[...] = reg.reshape(IMGS, N_PER_IMG, 2 * HW)


def kernel(boxes, gt_boxes, match_pos_flag, match_gt_id):
    B, N = boxes.shape[:2]
    KPS = 1
    BN = B * N

    flag = match_pos_flag.astype(jnp.int32).reshape(B, N, 1)
    gid = match_gt_id.astype(jnp.int32).reshape(B, N, 1)

    boxes_f = boxes.reshape(BN, 4)
    gt_f = gt_boxes.reshape(B * G_PER_IMG, 8)
    flag1 = match_pos_flag.astype(jnp.int32).reshape(BN)
    gid1 = match_gt_id.astype(jnp.int32).reshape(BN)
    mesh = plsc.VectorSubcoreMesh(core_axis_name="c", subcore_axis_name="s")
    clsw, regw = pl.kernel(
        _weights_sc,
        out_type=(
            jax.ShapeDtypeStruct((B, N, HW), jnp.float32),
            jax.ShapeDtypeStruct((B, N, 2 * HW), jnp.float32),
        ),
        mesh=mesh,
        scratch_types=[
            pltpu.VMEM((_RPT,), jnp.float32),
            pltpu.VMEM((_RPT,), jnp.float32),
            pltpu.VMEM((_RPT,), jnp.float32),
            pltpu.VMEM((_RPT,), jnp.float32),
            pltpu.VMEM((_RPT,), jnp.int32),
            pltpu.VMEM((_RPT,), jnp.int32),
            pltpu.VMEM((G_PER_IMG,), jnp.float32),
            pltpu.VMEM((G_PER_IMG,), jnp.float32),
            pltpu.VMEM((G_PER_IMG,), jnp.float32),
            pltpu.VMEM((_RPT, HW), jnp.float32),
            pltpu.VMEM((_RPT, 2 * HW), jnp.float32),
        ],
    )(boxes_f[:, 0], boxes_f[:, 1], boxes_f[:, 2], boxes_f[:, 3],
      flag1, gid1, gt_f[:, 4], gt_f[:, 5], gt_f[:, 6])

    grid = (B // IMGS,)
    out_shapes = (
        jax.ShapeDtypeStruct((B, N, HW), jnp.float32),
        jax.ShapeDtypeStruct((B, N, 2 * HW), jnp.float32),
    )
    in_specs = [
        pl.BlockSpec((IMGS, N, 4), lambda i: (i, 0, 0)),
        pl.BlockSpec((IMGS, 64, 8), lambda i: (i, 0, 0)),
        pl.BlockSpec((IMGS, N, 1), lambda i: (i, 0, 0)),
        pl.BlockSpec((IMGS, N, 1), lambda i: (i, 0, 0)),
    ]
    out_specs = (
        pl.BlockSpec((IMGS, N, HW), lambda i: (i, 0, 0)),
        pl.BlockSpec((IMGS, N, 2 * HW), lambda i: (i, 0, 0)),
    )
    cls, reg = pl.pallas_call(
        _label_kernel,
        grid=grid,
        in_specs=in_specs,
        out_specs=out_specs,
        out_shape=out_shapes,
    )(boxes, gt_boxes, flag, gid)

    return (cls.reshape(B, N, KPS, FEAT_H, FEAT_W),
            clsw.reshape(B, N, KPS, FEAT_H, FEAT_W),
            reg.reshape(B, N, 2 * KPS, FEAT_H, FEAT_W),
            regw.reshape(B, N, 2 * KPS, FEAT_H, FEAT_W))


# speedup vs baseline: 1.0019x; 1.0019x over previous
import jax
import jax.numpy as jnp
from jax import lax
from jax.experimental import pallas as pl
from jax.experimental.pallas import tpu as pltpu
from jax.experimental.pallas import tpu_sc as plsc

FEAT_H = 16
FEAT_W = 16
HW = FEAT_H * FEAT_W
GAUSS_TH = 0.6
EXPAND = 1.0
SIGMA = 1.6
BIN_OFF = 0.5
RADIUS = 1.0
NEG_LOG_TH = 0.5108255840295616
TWO_SIG2 = 2.0 * SIGMA ** 2
IMGS = 2
N_PER_IMG = 512
G_PER_IMG = 64

_NTILES = 32
_RPT = 64


def _take(v, idx):
    return lax.gather(
        v, idx[:, None],
        lax.GatherDimensionNumbers(offset_dims=(), collapsed_slice_dims=(0,),
                                   start_index_map=(0,)),
        (1,), mode=lax.GatherScatterMode.PROMISE_IN_BOUNDS)


def _weights_sc(x1_h, y1_h, x2_h, y2_h, flag_h, gid_h, kx_h, ky_h, kv_h,
                clsw_h, regw_h,
                x1_v, y1_v, x2_v, y2_v, flag_v, gid_v, kx_v, ky_v, kv_v,
                clsw_v, regw_v):
    wid = lax.axis_index("s") * 2 + lax.axis_index("c")
    base = wid * _RPT
    img = wid // (N_PER_IMG // _RPT)
    pltpu.sync_copy(x1_h.at[pl.ds(base, _RPT)], x1_v)
    pltpu.sync_copy(y1_h.at[pl.ds(base, _RPT)], y1_v)
    pltpu.sync_copy(x2_h.at[pl.ds(base, _RPT)], x2_v)
    pltpu.sync_copy(y2_h.at[pl.ds(base, _RPT)], y2_v)
    pltpu.sync_copy(flag_h.at[pl.ds(base, _RPT)], flag_v)
    pltpu.sync_copy(gid_h.at[pl.ds(base, _RPT)], gid_v)
    pltpu.sync_copy(kx_h.at[pl.ds(img * G_PER_IMG, G_PER_IMG)], kx_v)
    pltpu.sync_copy(ky_h.at[pl.ds(img * G_PER_IMG, G_PER_IMG)], ky_v)
    pltpu.sync_copy(kv_h.at[pl.ds(img * G_PER_IMG, G_PER_IMG)], kv_v)

    iota16 = lax.broadcasted_iota(jnp.int32, (16,), 0)
    zero16 = jnp.zeros((16,), jnp.float32)
    binc05 = iota16.astype(jnp.float32) + BIN_OFF

    kxc = [kx_v[pl.ds(c * 16, 16)] for c in range(4)]
    kyc = [ky_v[pl.ds(c * 16, 16)] for c in range(4)]
    kvc = [kv_v[pl.ds(c * 16, 16)] for c in range(4)]

    def tab_gather(chunks, lo, hi):
        acc = zero16
        for c in range(4):
            acc = jnp.where(hi == c, _take(chunks[c], lo), acc)
        return acc

    for g in range(_RPT // 16):
        sl = pl.ds(g * 16, 16)
        x1 = x1_v[sl]
        y1 = y1_v[sl]
        x2 = x2_v[sl]
        y2 = y2_v[sl]
        cx = (x1 + x2) * 0.5
        cy = (y1 + y2) * 0.5
        w = (x2 - x1 + 1.0) * EXPAND
        h = (y2 - y1 + 1.0) * EXPAND
        bx1 = cx - (w - 1.0) * 0.5
        by1 = cy - (h - 1.0) * 0.5
        bx2 = cx + (w - 1.0) * 0.5
        by2 = cy + (h - 1.0) * 0.5

        gid16 = gid_v[sl]
        lo = gid16 & 15
        hi = lax.shift_right_logical(gid16, 4)
        kx = tab_gather(kxc, lo, hi)
        ky = tab_gather(kyc, lo, hi)
        kv = tab_gather(kvc, lo, hi)

        sx = FEAT_W / (bx2 - bx1 + 1.0)
        sy = FEAT_H / (by2 - by1 + 1.0)
        x0v = (kx - bx1) * sx
        y0v = (ky - by1) * sy
        pvv = jnp.where((flag_v[sl] > 0) & (kv != 0.0), zero16 + 1.0, zero16)

        for p in range(16):
            sp = iota16 * 0 + p
            bx0 = _take(x0v, sp)
            by0 = _take(y0v, sp)
            pvb = _take(pvv, sp)
            dxv = binc05 - bx0
            argx = dxv * dxv / TWO_SIG2
            dyv = binc05 - by0
            ayv = dyv * dyv / TWO_SIG2
            row = g * 16 + p
            macc = zero16
            for j in range(FEAT_H):
                argj = argx + _take(ayv, iota16 * 0 + j)
                rw = jnp.where(argj <= NEG_LOG_TH, pvb, zero16)
                macc = jnp.maximum(macc, rw)
                regw_v[row, pl.ds(j * 16, 16)] = rw
                regw_v[row, pl.ds(HW + j * 16, 16)] = rw
            for s in (1, 2, 4, 8):
                macc = jnp.maximum(macc, _take(macc, iota16 ^ s))
            for j in range(FEAT_H):
                clsw_v[row, pl.ds(j * 16, 16)] = macc

    n0 = (wid % (N_PER_IMG // _RPT)) * _RPT
    pltpu.sync_copy(clsw_v, clsw_h.at[img, pl.ds(n0, _RPT)])
    pltpu.sync_copy(regw_v, regw_h.at[img, pl.ds(n0, _RPT)])


def _label_kernel(boxes_ref, gt_ref, flag_ref, gid_ref, cls_ref, reg_ref):
    rows = IMGS * N_PER_IMG
    ng = IMGS * G_PER_IMG
    boxes = boxes_ref[...].reshape(rows, 4)
    gt = gt_ref[...].reshape(ng, 8)
    flag = flag_ref[...].reshape(rows, 1)
    gid = gid_ref[...].reshape(rows, 1)

    goff = (lax.broadcasted_iota(jnp.int32, (rows, 1), 0)
            // N_PER_IMG) * G_PER_IMG
    gslot = gid + goff
    onehot = (gslot == lax.broadcasted_iota(jnp.int32, (rows, ng), 1)
              ).astype(jnp.float32)
    matched = jnp.dot(onehot, gt, preferred_element_type=jnp.float32,
                      precision=lax.Precision.HIGHEST)

    x1 = boxes[:, 0:1]
    y1 = boxes[:, 1:2]
    x2 = boxes[:, 2:3]
    y2 = boxes[:, 3:4]
    cx = (x1 + x2) * 0.5
    cy = (y1 + y2) * 0.5
    w = (x2 - x1 + 1.0) * EXPAND
    h = (y2 - y1 + 1.0) * EXPAND
    bx1 = cx - (w - 1.0) * 0.5
    by1 = cy - (h - 1.0) * 0.5
    bx2 = cx + (w - 1.0) * 0.5
    by2 = cy + (h - 1.0) * 0.5

    kx = matched[:, 4:5]
    ky = matched[:, 5:6]
    kv = matched[:, 6:7]

    sx = FEAT_W / (bx2 - bx1 + 1.0)
    sy = FEAT_H / (by2 - by1 + 1.0)
    x0 = (kx - bx1) * sx
    y0 = (ky - by1) * sy

    col = lax.broadcasted_iota(jnp.int32, (rows, HW), 1)
    bin_x = (col % FEAT_W).astype(jnp.float32)
    bin_y = (col // FEAT_W).astype(jnp.float32)

    dx = bin_x + BIN_OFF - x0
    dy = bin_y + BIN_OFF - y0
    inv2s2 = 1.0 / TWO_SIG2
    arg = dx * dx * inv2s2 + dy * dy * inv2s2
    score = jnp.exp(-arg)
    keep = arg <= NEG_LOG_TH

    vis = kv != 0.0
    pos = flag > 0
    active = pos & vis & jnp.any(keep, axis=-1, keepdims=True)

    cls_ref[...] = jnp.where(active, score, -1.0).reshape(IMGS, N_PER_IMG, HW)

    m = pos & vis & keep
    off_x = (x0 - bin_x) / RADIUS
    off_y = (y0 - bin_y) / RADIUS
    zeros = jnp.zeros_like(score)
    reg = jnp.concatenate([jnp.where(m, off_x, zeros),
                           jnp.where(m, off_y, zeros)], axis=1)
    reg_ref[...] = reg.reshape(IMGS, N_PER_IMG, 2 * HW)


def kernel(boxes, gt_boxes, match_pos_flag, match_gt_id):
    B, N = boxes.shape[:2]
    KPS = 1
    BN = B * N

    flag = match_pos_flag.astype(jnp.int32).reshape(B, N, 1)
    gid = match_gt_id.astype(jnp.int32).reshape(B, N, 1)

    boxes_f = boxes.reshape(BN, 4)
    gt_f = gt_boxes.reshape(B * G_PER_IMG, 8)
    flag1 = match_pos_flag.astype(jnp.int32).reshape(BN)
    gid1 = match_gt_id.astype(jnp.int32).reshape(BN)
    mesh = plsc.VectorSubcoreMesh(core_axis_name="c", subcore_axis_name="s", num_cores=2)
    clsw, regw = pl.kernel(
        _weights_sc,
        out_type=(
            jax.ShapeDtypeStruct((B, N, HW), jnp.float32),
            jax.ShapeDtypeStruct((B, N, 2 * HW), jnp.float32),
        ),
        mesh=mesh,
        scratch_types=[
            pltpu.VMEM((_RPT,), jnp.float32),
            pltpu.VMEM((_RPT,), jnp.float32),
            pltpu.VMEM((_RPT,), jnp.float32),
            pltpu.VMEM((_RPT,), jnp.float32),
            pltpu.VMEM((_RPT,), jnp.int32),
            pltpu.VMEM((_RPT,), jnp.int32),
            pltpu.VMEM((G_PER_IMG,), jnp.float32),
            pltpu.VMEM((G_PER_IMG,), jnp.float32),
            pltpu.VMEM((G_PER_IMG,), jnp.float32),
            pltpu.VMEM((_RPT, HW), jnp.float32),
            pltpu.VMEM((_RPT, 2 * HW), jnp.float32),
        ],
    )(boxes_f[:, 0], boxes_f[:, 1], boxes_f[:, 2], boxes_f[:, 3],
      flag1, gid1, gt_f[:, 4], gt_f[:, 5], gt_f[:, 6])

    grid = (B // IMGS,)
    out_shapes = (
        jax.ShapeDtypeStruct((B, N, HW), jnp.float32),
        jax.ShapeDtypeStruct((B, N, 2 * HW), jnp.float32),
    )
    in_specs = [
        pl.BlockSpec((IMGS, N, 4), lambda i: (i, 0, 0)),
        pl.BlockSpec((IMGS, 64, 8), lambda i: (i, 0, 0)),
        pl.BlockSpec((IMGS, N, 1), lambda i: (i, 0, 0)),
        pl.BlockSpec((IMGS, N, 1), lambda i: (i, 0, 0)),
    ]
    out_specs = (
        pl.BlockSpec((IMGS, N, HW), lambda i: (i, 0, 0)),
        pl.BlockSpec((IMGS, N, 2 * HW), lambda i: (i, 0, 0)),
    )
    cls, reg = pl.pallas_call(
        _label_kernel,
        grid=grid,
        in_specs=in_specs,
        out_specs=out_specs,
        out_shape=out_shapes,
    )(boxes, gt_boxes, flag, gid)

    return (cls.reshape(B, N, KPS, FEAT_H, FEAT_W),
            clsw.reshape(B, N, KPS, FEAT_H, FEAT_W),
            reg.reshape(B, N, 2 * KPS, FEAT_H, FEAT_W),
            regw.reshape(B, N, 2 * KPS, FEAT_H, FEAT_W))
